# baseline (speedup 1.0000x reference)
import functools

import jax
import jax.numpy as jnp
from jax import lax
from jax.experimental import pallas as pl
from jax.experimental.pallas import tpu as pltpu
from jax.experimental.pallas import tpu_sc as plsc

D = 64
NUM_ROWS = 1000000
B = 4096 * 200
NC, NS = 2, 16
NW = NC * NS
B_PER_W = B // NW
CHUNK = 256
NBUF = 2
N_CHUNKS = B_PER_W // CHUNK
N_GROUPS = N_CHUNKS // NBUF

_mesh = plsc.VectorSubcoreMesh(core_axis_name="c", subcore_axis_name="s")


@functools.partial(
    pl.kernel,
    mesh=_mesh,
    out_type=jax.ShapeDtypeStruct((2 * B, D), jnp.float32),
    scratch_types=[
        pltpu.VMEM((2 * B_PER_W,), jnp.int32),
        pltpu.VMEM((NBUF, 2 * CHUNK, D), jnp.float32),
        pltpu.SemaphoreType.DMA,
        [pltpu.SemaphoreType.DMA] * NBUF,
        [pltpu.SemaphoreType.DMA] * NBUF,
    ],
    compiler_params=pltpu.CompilerParams(use_tc_tiling_on_sc=False),
)
def _emb_lookup(idx2_hbm, table_hbm, out_hbm, idx2_v, rows_v, isem,
                gsems, ssems):
    wid = lax.axis_index("s") * NC + lax.axis_index("c")
    base = wid * B_PER_W

    pltpu.async_copy(
        idx2_hbm.at[pl.ds(2 * base, 2 * B_PER_W)], idx2_v, isem
    ).wait()

    def idx_slice(chunk):
        return idx2_v.at[pl.ds(chunk * 2 * CHUNK, 2 * CHUNK)]

    def start_gather(chunk, b):
        pltpu.async_copy(table_hbm.at[idx_slice(chunk)], rows_v.at[b], gsems[b])

    def wait_gather(chunk, b):
        pltpu.make_async_copy(
            table_hbm.at[idx_slice(chunk)], rows_v.at[b], gsems[b]
        ).wait()

    def out_slice(chunk):
        return out_hbm.at[pl.ds(2 * (base + chunk * CHUNK), 2 * CHUNK)]

    def start_store(chunk, b):
        pltpu.async_copy(rows_v.at[b], out_slice(chunk), ssems[b])

    def wait_store(chunk, b):
        pltpu.make_async_copy(rows_v.at[b], out_slice(chunk), ssems[b]).wait()

    for b in range(NBUF):
        start_gather(b, b)

    def group(g, carry):
        c0 = g * NBUF
        for b in range(NBUF):
            wait_gather(c0 + b, b)
            start_store(c0 + b, b)
        for b in range(NBUF):
            wait_store(c0 + b, b)
            start_gather(c0 + NBUF + b, b)
        return carry

    lax.fori_loop(0, N_GROUPS - 1, group, 0)

    c0 = (N_GROUPS - 1) * NBUF
    for b in range(NBUF):
        wait_gather(c0 + b, b)
        start_store(c0 + b, b)
    for b in range(NBUF):
        wait_store(c0 + b, b)


def kernel(token_ids, embedding):
    flat = jnp.repeat(token_ids.reshape(-1).astype(jnp.int32), 2)
    table2 = jax.lax.optimization_barrier(embedding.reshape(NUM_ROWS // 2, 2 * D))
    table = table2.reshape(NUM_ROWS, D)
    out = _emb_lookup(flat, table)
    sliced = out.reshape(B, 2 * D)[:, :D]
    return sliced.reshape(token_ids.shape + (D,))


# speedup vs baseline: 1.0021x; 1.0021x over previous
import functools

import jax
import jax.numpy as jnp
from jax import lax
from jax.experimental import pallas as pl
from jax.experimental.pallas import tpu as pltpu
from jax.experimental.pallas import tpu_sc as plsc

D = 64
NUM_ROWS = 1000000
B = 4096 * 200
NC, NS = 2, 16
NW = NC * NS
B_PER_W = B // NW
CHUNK = 256
NBUF = 2
N_CHUNKS = B_PER_W // CHUNK
N_GROUPS = N_CHUNKS // NBUF

_mesh = plsc.VectorSubcoreMesh(core_axis_name="c", subcore_axis_name="s")


@functools.partial(
    pl.kernel,
    mesh=_mesh,
    out_type=jax.ShapeDtypeStruct((2 * B, D), jnp.float32),
    scratch_types=[
        pltpu.VMEM((2 * B_PER_W,), jnp.int32),
        pltpu.VMEM((NBUF, 2 * CHUNK, D), jnp.float32),
        pltpu.SemaphoreType.DMA,
        [pltpu.SemaphoreType.DMA] * NBUF,
        [pltpu.SemaphoreType.DMA] * NBUF,
    ],
    compiler_params=pltpu.CompilerParams(use_tc_tiling_on_sc=False),
)
def _emb_lookup(idx2_hbm, table_hbm, out_hbm, idx2_v, rows_v, isem,
                gsems, ssems):
    wid = lax.axis_index("s") * NC + lax.axis_index("c")
    base = wid * B_PER_W

    pltpu.async_copy(
        idx2_hbm.at[pl.ds(2 * base, 2 * B_PER_W)], idx2_v, isem
    ).wait()

    def idx_slice(chunk):
        return idx2_v.at[pl.ds(chunk * 2 * CHUNK, 2 * CHUNK)]

    def start_gather(chunk, b):
        pltpu.async_copy(table_hbm.at[idx_slice(chunk)], rows_v.at[b], gsems[b])

    def wait_gather(chunk, b):
        pltpu.make_async_copy(
            table_hbm.at[idx_slice(chunk)], rows_v.at[b], gsems[b]
        ).wait()

    def out_slice(chunk):
        return out_hbm.at[pl.ds(2 * (base + chunk * CHUNK), 2 * CHUNK)]

    def start_store(chunk, b):
        pltpu.async_copy(rows_v.at[b], out_slice(chunk), ssems[b])

    def wait_store(chunk, b):
        pltpu.make_async_copy(rows_v.at[b], out_slice(chunk), ssems[b]).wait()

    for b in range(NBUF):
        start_gather(b, b)

    def group(g, carry):
        c0 = g * NBUF
        for b in range(NBUF):
            wait_gather(c0 + b, b)
            start_store(c0 + b, b)
        for b in range(NBUF):
            wait_store(c0 + b, b)
            start_gather(c0 + NBUF + b, b)
        return carry

    lax.fori_loop(0, N_GROUPS - 1, group, 0)

    c0 = (N_GROUPS - 1) * NBUF
    for b in range(NBUF):
        wait_gather(c0 + b, b)
        start_store(c0 + b, b)
    for b in range(NBUF):
        wait_store(c0 + b, b)


def kernel(token_ids, embedding):
    ids = token_ids.reshape(-1).astype(jnp.int32)
    two_b = jax.lax.optimization_barrier(jnp.broadcast_to(ids[None, :], (2, B)))
    flat = two_b.T.reshape(2 * B)
    table2 = jax.lax.optimization_barrier(embedding.reshape(NUM_ROWS // 2, 2 * D))
    table = table2.reshape(NUM_ROWS, D)
    out = _emb_lookup(flat, table)
    sliced = out.reshape(B, 2 * D)[:, :D]
    return sliced.reshape(token_ids.shape + (D,))


# speedup vs baseline: 1.3752x; 1.3723x over previous
import functools

import jax
import jax.numpy as jnp
from jax import lax
from jax.experimental import pallas as pl
from jax.experimental.pallas import tpu as pltpu
from jax.experimental.pallas import tpu_sc as plsc

D = 64
NUM_ROWS = 1000000
B = 4096 * 200
NC, NS = 2, 16
NW = NC * NS
B_PER_W = B // NW
CHUNK = 128
NBUF = 2
N_CHUNKS = B_PER_W // CHUNK
N_GROUPS = N_CHUNKS // NBUF

_mesh = plsc.VectorSubcoreMesh(core_axis_name="c", subcore_axis_name="s")


@functools.partial(
    pl.kernel,
    mesh=_mesh,
    out_type=jax.ShapeDtypeStruct((2 * B, D), jnp.float32),
    scratch_types=[
        pltpu.VMEM((B_PER_W,), jnp.int32),
        pltpu.VMEM((2 * B_PER_W,), jnp.int32),
        pltpu.VMEM((NBUF, 2 * CHUNK, D), jnp.float32),
        pltpu.SemaphoreType.DMA,
        [pltpu.SemaphoreType.DMA] * NBUF,
        [pltpu.SemaphoreType.DMA] * NBUF,
    ],
    compiler_params=pltpu.CompilerParams(
        use_tc_tiling_on_sc=False, needs_layout_passes=False
    ),
)
def _emb_lookup(idx_hbm, table_hbm, out_hbm, idx_v, idx2_v, rows_v, isem,
                gsems, ssems):
    wid = lax.axis_index("s") * NC + lax.axis_index("c")
    base = wid * B_PER_W

    pltpu.async_copy(idx_hbm.at[pl.ds(base, B_PER_W)], idx_v, isem).wait()

    lanes = jax.lax.iota(jnp.int32, 16)

    def dup(i, carry):
        iv, pv = carry
        ids = plsc.load_gather(idx_v, [iv])
        plsc.store_scatter(idx2_v, [pv], ids)
        plsc.store_scatter(idx2_v, [pv + 1], ids)
        return iv + 16, pv + 32

    lax.fori_loop(0, B_PER_W // 16, dup, (lanes, lanes * 2))

    def idx_slice(chunk):
        return idx2_v.at[pl.ds(chunk * 2 * CHUNK, 2 * CHUNK)]

    def start_gather(chunk, b):
        pltpu.async_copy(table_hbm.at[idx_slice(chunk)], rows_v.at[b], gsems[b])

    def wait_gather(chunk, b):
        pltpu.make_async_copy(
            table_hbm.at[idx_slice(chunk)], rows_v.at[b], gsems[b]
        ).wait()

    def out_slice(chunk):
        return out_hbm.at[pl.ds(2 * (base + chunk * CHUNK), 2 * CHUNK)]

    def start_store(chunk, b):
        pltpu.async_copy(rows_v.at[b], out_slice(chunk), ssems[b])

    def wait_store(chunk, b):
        pltpu.make_async_copy(rows_v.at[b], out_slice(chunk), ssems[b]).wait()

    for b in range(NBUF):
        start_gather(b, b)

    def group(g, carry):
        c0 = g * NBUF
        for b in range(NBUF):
            wait_gather(c0 + b, b)
            start_store(c0 + b, b)
        for b in range(NBUF):
            wait_store(c0 + b, b)
            start_gather(c0 + NBUF + b, b)
        return carry

    lax.fori_loop(0, N_GROUPS - 1, group, 0)

    c0 = (N_GROUPS - 1) * NBUF
    for b in range(NBUF):
        wait_gather(c0 + b, b)
        start_store(c0 + b, b)
    for b in range(NBUF):
        wait_store(c0 + b, b)


def kernel(token_ids, embedding):
    flat = token_ids.reshape(-1).astype(jnp.int32)
    table2 = jax.lax.optimization_barrier(embedding.reshape(NUM_ROWS // 2, 2 * D))
    table = table2.reshape(NUM_ROWS, D)
    out = _emb_lookup(flat, table)
    sliced = out.reshape(B, 2 * D)[:, :D]
    return sliced.reshape(token_ids.shape + (D,))


# speedup vs baseline: 1.5093x; 1.0976x over previous
import functools

import jax
import jax.numpy as jnp
from jax import lax
from jax.experimental import pallas as pl
from jax.experimental.pallas import tpu as pltpu
from jax.experimental.pallas import tpu_sc as plsc

D = 64
NUM_ROWS = 1000000
B = 4096 * 200
NC, NS = 2, 16
NW = NC * NS
B_PER_W = B // NW
CHUNK = 256
NBUF = 2
N_CHUNKS = B_PER_W // CHUNK
N_GROUPS = N_CHUNKS // NBUF

N_BLOCKS_FULL = NUM_ROWS // 128
TAIL_ROWS = NUM_ROWS - N_BLOCKS_FULL * 128
BASE_BLOCKS = N_BLOCKS_FULL // NW
EXTRA_W = N_BLOCKS_FULL - BASE_BLOCKS * NW

_mesh = plsc.VectorSubcoreMesh(core_axis_name="c", subcore_axis_name="s")

_LANES = None


@functools.partial(
    pl.kernel,
    mesh=_mesh,
    out_type=jax.ShapeDtypeStruct((NUM_ROWS * 128,), jnp.float32),
    scratch_types=[
        [pltpu.VMEM((8, 8, 128), jnp.float32)] * 2,
        [pltpu.VMEM((128 * 128,), jnp.float32)] * 2,
        [pltpu.SemaphoreType.DMA] * 2,
        [pltpu.SemaphoreType.DMA] * 2,
    ],
    compiler_params=pltpu.CompilerParams(
        use_tc_tiling_on_sc=True, needs_layout_passes=False
    ),
)
def _transpose_pad(t_hbm, tail_hbm, out_hbm, slabs, obufs, isems, osems):
    wid = lax.axis_index("s") * NC + lax.axis_index("c")
    n_per_w = 246
    blk0 = wid * n_per_w

    lanes = lax.iota(jnp.int32, 16)
    tvecs = [(s * 16 + lanes) // 8 for s in range(4)]
    ccvecs = [(s * 16 + lanes) % 8 for s in range(4)]

    def start_in(blk, b):
        i0 = pl.multiple_of(blk * 128, 128)
        for t in range(8):
            pltpu.async_copy(
                t_hbm.at[pl.ds(8 * t, 8), pl.ds(i0, 128)],
                slabs[b].at[t],
                isems[b],
            )

    def wait_in(blk, b):
        i0 = pl.multiple_of(blk * 128, 128)
        for t in range(8):
            pltpu.make_async_copy(
                t_hbm.at[pl.ds(8 * t, 8), pl.ds(i0, 128)],
                slabs[b].at[t],
                isems[b],
            ).wait()

    def start_out(blk, b):
        pltpu.async_copy(
            obufs[b], out_hbm.at[pl.ds(blk * 16384, 16384)], osems[b]
        )

    def wait_out(blk, b):
        pltpu.make_async_copy(
            obufs[b], out_hbm.at[pl.ds(blk * 16384, 16384)], osems[b]
        ).wait()

    def compute(b, n_rows):
        zero = lanes - lanes

        def chunk8(u, sv):
            for _ in range(8):
                for s in range(4):
                    v = plsc.load_gather(slabs[b], [tvecs[s], ccvecs[s], sv])
                    plsc.store_scatter(
                        obufs[b], [sv * 128 + (s * 16 + lanes)], v
                    )
                sv = sv + 1
            return sv

        lax.fori_loop(0, n_rows // 8, chunk8, zero)

    def valid(blk):
        return blk < N_BLOCKS_FULL

    def guarded_start_in(blk, b):
        @pl.when(valid(blk))
        def _():
            start_in(blk, b)

    guarded_start_in(blk0, 0)
    guarded_start_in(blk0 + 1, 1)

    def body(j, first):
        for b in range(2):
            blk = blk0 + j + b

            @pl.when(valid(blk))
            def _():
                wait_in(blk, b)

            @pl.when(jnp.logical_and(first == 0, valid(blk - 2)))
            def _():
                wait_out(blk - 2, b)

            @pl.when(valid(blk))
            def _():
                compute(b, 128)
                start_out(blk, b)

            guarded_start_in(blk + 2, b)
        return first * 0

    lax.fori_loop(0, n_per_w // 2, lambda g, c: body(2 * g, c), 1)

    for b in range(2):
        blk = blk0 + n_per_w - 2 + b

        @pl.when(valid(blk))
        def _():
            wait_out(blk, b)

    @pl.when(wid == NW - 1)
    def _():
        i0 = N_BLOCKS_FULL * 128
        for t in range(8):
            pltpu.async_copy(
                tail_hbm.at[pl.ds(8 * t, 8)], slabs[0].at[t], isems[0]
            )
        for t in range(8):
            pltpu.make_async_copy(
                tail_hbm.at[pl.ds(8 * t, 8)], slabs[0].at[t], isems[0]
            ).wait()
        compute(0, TAIL_ROWS)
        pltpu.async_copy(
            obufs[0].at[pl.ds(0, TAIL_ROWS * 128)],
            out_hbm.at[pl.ds(i0 * 128, TAIL_ROWS * 128)],
            osems[0],
        )
        pltpu.make_async_copy(
            obufs[0].at[pl.ds(0, TAIL_ROWS * 128)],
            out_hbm.at[pl.ds(i0 * 128, TAIL_ROWS * 128)],
            osems[0],
        ).wait()


@functools.partial(
    pl.kernel,
    mesh=_mesh,
    out_type=jax.ShapeDtypeStruct((B, 2 * D), jnp.float32),
    scratch_types=[
        pltpu.VMEM((B_PER_W,), jnp.int32),
        pltpu.VMEM((NBUF, CHUNK, 2 * D), jnp.float32),
        pltpu.SemaphoreType.DMA,
        [pltpu.SemaphoreType.DMA] * NBUF,
        [pltpu.SemaphoreType.DMA] * NBUF,
    ],
    compiler_params=pltpu.CompilerParams(use_tc_tiling_on_sc=False),
)
def _emb_lookup(idx_hbm, table_hbm, out_hbm, idx_v, rows_v, isem, gsems,
                ssems):
    wid = lax.axis_index("s") * NC + lax.axis_index("c")
    base = wid * B_PER_W

    pltpu.async_copy(idx_hbm.at[pl.ds(base, B_PER_W)], idx_v, isem).wait()

    def idx_slice(chunk):
        return idx_v.at[pl.ds(chunk * CHUNK, CHUNK)]

    def start_gather(chunk, b):
        pltpu.async_copy(table_hbm.at[idx_slice(chunk)], rows_v.at[b], gsems[b])

    def wait_gather(chunk, b):
        pltpu.make_async_copy(
            table_hbm.at[idx_slice(chunk)], rows_v.at[b], gsems[b]
        ).wait()

    def out_slice(chunk):
        return out_hbm.at[pl.ds(base + chunk * CHUNK, CHUNK)]

    def start_store(chunk, b):
        pltpu.async_copy(rows_v.at[b], out_slice(chunk), ssems[b])

    def wait_store(chunk, b):
        pltpu.make_async_copy(rows_v.at[b], out_slice(chunk), ssems[b]).wait()

    for b in range(NBUF):
        start_gather(b, b)

    def group(g, carry):
        c0 = g * NBUF
        for b in range(NBUF):
            wait_gather(c0 + b, b)
            start_store(c0 + b, b)
        for b in range(NBUF):
            wait_store(c0 + b, b)
            start_gather(c0 + NBUF + b, b)
        return carry

    lax.fori_loop(0, N_GROUPS - 1, group, 0)

    c0 = (N_GROUPS - 1) * NBUF
    for b in range(NBUF):
        wait_gather(c0 + b, b)
        start_store(c0 + b, b)
    for b in range(NBUF):
        wait_store(c0 + b, b)


def kernel(token_ids, embedding):
    flat = token_ids.reshape(-1).astype(jnp.int32)
    table128 = jnp.pad(embedding, ((0, 0), (0, D)))
    out = _emb_lookup(flat, table128)
    sliced = out[:, :D]
    return sliced.reshape(token_ids.shape + (D,))
